# baseline (speedup 1.0000x reference)
import functools

import jax
import jax.numpy as jnp
from jax import lax
from jax.experimental import pallas as pl
from jax.experimental.pallas import tpu as pltpu
from jax.experimental.pallas import tpu_sc as plsc


def _conv_body(*args, KD, KH, KW, Wp, M, dz0, o0, act, windowed, precision):
    xrs, (wr, br, outr) = args[:-3], args[-3:]
    d = pl.program_id(1)
    pieces = []
    for kd in range(KD):
        for kh in range(KH):
            for kw in range(KW):
                o = o0 + kh * Wp + kw
                if windowed:
                    pieces.append(xrs[kd][0, 0, pl.ds(o, M), :])
                else:
                    pieces.append(xrs[0][0, d + dz0 + kd, pl.ds(o, M), :])
    patch = jnp.concatenate(pieces, axis=-1)
    y = jnp.dot(patch, wr[...], preferred_element_type=jnp.float32,
                precision=precision)
    y = y + br[...]
    if act == "relu":
        y = jnp.maximum(y, 0.0)
    elif act == "sigmoid":
        y = jax.nn.sigmoid(y)
    outr[0, 0] = y


def _conv_stage(xp, wmat, bias, *, KD, KH, KW, Wp, H_out, D_out, act,
                dz0=0, o0=0, windowed=False, precision=None,
                compute_dtype=None):
    if compute_dtype is not None:
        xp = xp.astype(compute_dtype)
        wmat = wmat.astype(compute_dtype)
    N, Dp, HWpad, Cin = xp.shape
    Ktot, Cout = wmat.shape
    M = H_out * Wp
    body = functools.partial(_conv_body, KD=KD, KH=KH, KW=KW, Wp=Wp, M=M,
                             dz0=dz0, o0=o0, act=act, windowed=windowed,
                             precision=precision)
    if windowed:
        x_specs = [
            pl.BlockSpec((1, 1, HWpad, Cin),
                         lambda n, d, kd=kd: (n, d + dz0 + kd, 0, 0))
            for kd in range(KD)
        ]
        x_args = (xp,) * KD
    else:
        x_specs = [pl.BlockSpec((1, Dp, HWpad, Cin), lambda n, d: (n, 0, 0, 0))]
        x_args = (xp,)
    return pl.pallas_call(
        body,
        grid=(N, D_out),
        in_specs=x_specs + [
            pl.BlockSpec((Ktot, Cout), lambda n, d: (0, 0)),
            pl.BlockSpec((1, Cout), lambda n, d: (0, 0)),
        ],
        out_specs=pl.BlockSpec((1, 1, M, Cout), lambda n, d: (n, d, 0, 0)),
        out_shape=jax.ShapeDtypeStruct((N, D_out, M, Cout), jnp.float32),
    )(*x_args, wmat, bias)


def _pad_flat(x, pad):
    N, D, H, W, C = x.shape
    xp = jnp.pad(x, ((0, 0), (pad, pad), (pad, pad), (pad, pad), (0, 0)))
    Dp, Hp, Wpd = D + 2 * pad, H + 2 * pad, W + 2 * pad
    xp = xp.reshape(N, Dp, Hp * Wpd, C)
    return jnp.pad(xp, ((0, 0), (0, 0), (0, 8), (0, 0))), Wpd


def _unflat(y, H_out, W_out):
    N, D, M, C = y.shape
    Wp = M // H_out
    return y.reshape(N, D, H_out, Wp, C)[:, :, :, :W_out, :]


def _bn_fold(w_mat, conv_b, g, b, m, v, eps=1e-5):
    s = g / jnp.sqrt(v + eps)
    return w_mat * s[None, :], ((conv_b - m) * s + b)[None, :]


def _vq_body(zr, er, e2r, idxr, cntr, csumr, perpr, *, R, K, B, D):
    i = pl.program_id(0)
    z = zr[0]
    scores = jnp.dot(z, er[...], preferred_element_type=jnp.float32)
    z2 = jnp.sum(z * z, axis=1)
    t = (z2[:, None] + e2r[...]) - 2.0 * scores
    idx = jnp.argmin(t, axis=1).astype(jnp.int32)
    csum = jnp.sum(jnp.min(t, axis=1)).reshape(1, 1)
    idxr[0, 0] = idx
    onehot = (lax.broadcasted_iota(jnp.int32, (R, K), 1)
              == idx[:, None]).astype(jnp.float32)
    c = jnp.sum(onehot, axis=0)

    @pl.when(i == 0)
    def _():
        cntr[0] = c
        csumr[...] = csum

    @pl.when(i > 0)
    def _():
        cntr[0] += c
        csumr[...] = csumr[...] + csum

    @pl.when(i == pl.num_programs(0) - 1)
    def _():
        p = cntr[0] * (1.0 / B)
        perpr[...] = jnp.exp(-jnp.sum(p * jnp.log(p + 1e-10))).reshape(1, 1)
        csumr[...] = csumr[...] * (0.25 / (B * D))


def _vq_stage(flat, emb):
    B, D = flat.shape
    K = emb.shape[0]
    R = 2048
    e2 = jnp.sum(emb * emb, axis=1)[None, :]
    body = functools.partial(_vq_body, R=R, K=K, B=B, D=D)
    idx, cnt, csum, perp = pl.pallas_call(
        body,
        grid=(B // R,),
        in_specs=[
            pl.BlockSpec((1, R, D), lambda i: (i, 0, 0)),
            pl.BlockSpec((D, K), lambda i: (0, 0)),
            pl.BlockSpec((1, K), lambda i: (0, 0)),
        ],
        out_specs=[
            pl.BlockSpec((1, 1, R), lambda i: (i, 0, 0)),
            pl.BlockSpec((1, K), lambda i: (0, 0)),
            pl.BlockSpec((1, 1), lambda i: (0, 0)),
            pl.BlockSpec((1, 1), lambda i: (0, 0)),
        ],
        out_shape=[
            jax.ShapeDtypeStruct((B // R, 1, R), jnp.int32),
            jax.ShapeDtypeStruct((1, K), jnp.float32),
            jax.ShapeDtypeStruct((1, 1), jnp.float32),
            jax.ShapeDtypeStruct((1, 1), jnp.float32),
        ],
    )(flat.reshape(B // R, R, D), emb.T, e2)
    return idx.reshape(B), csum[0, 0], perp[0, 0]


def _quant_gather(emb, idx):
    emb = jnp.pad(emb, ((0, 0), (0, 128 - emb.shape[1])))
    B = idx.shape[0]
    D = emb.shape[1]
    info = plsc.get_sparse_core_info()
    NW = info.num_cores * info.num_subcores
    b_per_w = B // NW
    CH = 512
    n_ch = b_per_w // CH
    mesh = plsc.VectorSubcoreMesh(core_axis_name="c", subcore_axis_name="s")

    @functools.partial(
        pl.kernel, mesh=mesh,
        out_type=jax.ShapeDtypeStruct((B, D), jnp.float32),
        scratch_types=[
            pltpu.VMEM((CH,), jnp.int32),
            pltpu.VMEM((CH, D), jnp.float32),
            pltpu.SemaphoreType.DMA,
        ],
    )
    def gather_k(table_hbm, idx_hbm, out_hbm, idx_v, rows_v, sem):
        wid = lax.axis_index("s") * info.num_cores + lax.axis_index("c")
        base = wid * b_per_w
        for ch in range(n_ch):
            pltpu.sync_copy(idx_hbm.at[pl.ds(base + ch * CH, CH)], idx_v)
            pltpu.async_copy(table_hbm.at[idx_v], rows_v, sem).wait()
            pltpu.sync_copy(rows_v, out_hbm.at[pl.ds(base + ch * CH, CH)])

    return gather_k(emb, idx)


def kernel(x, p):
    N = x.shape[0]
    f32 = jnp.float32

    xp = jnp.pad(x[:, 0], ((0, 0), (1, 1), (1, 1), (1, 1)))
    xp = xp.reshape(N, 17, 2, 33, 2, 33, 2).transpose(0, 1, 3, 5, 2, 4, 6)
    xp = xp.reshape(N, 17, 33 * 33, 8)
    xp = jnp.pad(xp, ((0, 0), (0, 0), (0, 8), (0, 0)))
    w1 = p["ew1"].reshape(32, 2, 2, 2, 2, 2, 2).transpose(1, 3, 5, 2, 4, 6, 0)
    w1 = w1.reshape(64, 32)
    w1, b1 = _bn_fold(w1, p["eb1"], p["eg1"], p["ebt1"], p["em1"], p["ev1"])
    h = _conv_stage(xp, w1, b1, KD=2, KH=2, KW=2, Wp=33, H_out=32, D_out=16,
                    act="relu")
    h = _unflat(h, 32, 32)

    hp, Wp = _pad_flat(h, 1)
    w2 = p["ew2"].transpose(2, 3, 4, 1, 0).reshape(27 * 32, 64)
    w2, b2 = _bn_fold(w2, p["eb2"], p["eg2"], p["ebt2"], p["em2"], p["ev2"])
    h = _conv_stage(hp, w2, b2, KD=3, KH=3, KW=3, Wp=Wp, H_out=32, D_out=16,
                    act="relu")
    h = _unflat(h, 32, 32)

    hp, Wp = _pad_flat(h, 1)
    w3 = p["ew3"].transpose(2, 3, 4, 1, 0).reshape(27 * 64, 64)
    b3 = p["eb3"][None, :]
    z = _conv_stage(hp, w3, b3, KD=3, KH=3, KW=3, Wp=Wp, H_out=32, D_out=16,
                    act="none")
    z = _unflat(z, 32, 32)
    flat = z.reshape(-1, 64)

    idx, commitment, perplexity = _vq_stage(flat, p["emb"])

    quant = _quant_gather(p["emb"], idx)[:, :64]
    quant = quant.reshape(N, 16, 32, 32, 64)

    hp, Wp = _pad_flat(quant, 1)
    dw1 = p["dw1"].transpose(2, 3, 4, 1, 0).reshape(27 * 64, 64)
    dw1, db1 = _bn_fold(dw1, p["db1"], p["dg1"], p["dbt1"], p["dm1"], p["dv1"])
    h = _conv_stage(hp, dw1, db1, KD=3, KH=3, KW=3, Wp=Wp, H_out=32, D_out=16,
                    act="relu")
    h = _unflat(h, 32, 32)

    hp, Wp = _pad_flat(h, 1)
    f = jnp.flip(p["dwt"], axis=(2, 3, 4))
    f = f.reshape(64, 32, 2, 2, 2, 2, 2, 2).transpose(3, 5, 7, 2, 4, 6, 0, 1)
    f = f.reshape(8, 8 * 64, 32)
    s2 = p["dg2"] / jnp.sqrt(p["dv2"] + 1e-5)
    bt = ((p["dtb"] - p["dm2"]) * s2 + p["dbt2"])[None, :]
    phases = []
    for pz in range(2):
        for py in range(2):
            for px in range(2):
                ph = (pz * 2 + py) * 2 + px
                wp_ = f[ph] * s2[None, :]
                o = _conv_stage(hp, wp_, bt, KD=2, KH=2, KW=2, Wp=Wp,
                                H_out=32, D_out=16, act="relu",
                                dz0=pz, o0=py * Wp + px)
                phases.append(_unflat(o, 32, 32))
    ph = jnp.stack(phases).reshape(2, 2, 2, N, 16, 32, 32, 32)
    h = ph.transpose(3, 4, 0, 5, 1, 6, 2, 7).reshape(N, 32, 64, 64, 32)

    hp, Wp = _pad_flat(h, 1)
    dw3 = p["dw3"].transpose(2, 3, 4, 1, 0).reshape(27 * 32, 1)
    dw3 = jnp.pad(dw3, ((0, 0), (0, 7)))
    db3 = jnp.pad(p["db3"], (0, 7))[None, :]
    y = _conv_stage(hp, dw3, db3, KD=3, KH=3, KW=3, Wp=Wp, H_out=64, D_out=32,
                    act="sigmoid", windowed=True)
    y = _unflat(y, 64, 64)[..., :1]
    x_recon = y.transpose(0, 4, 1, 2, 3).astype(f32)

    return x_recon, commitment.astype(f32), perplexity.astype(f32)


# speedup vs baseline: 1.2231x; 1.2231x over previous
import functools

import jax
import jax.numpy as jnp
from jax import lax
from jax.experimental import pallas as pl
from jax.experimental.pallas import tpu as pltpu
from jax.experimental.pallas import tpu_sc as plsc


def _conv_body(*args, KD, KH, KW, Wp, M, dz0, o0, act, windowed, precision):
    xrs, (wr, br, outr) = args[:-3], args[-3:]
    d = pl.program_id(1)
    pieces = []
    for kd in range(KD):
        for kh in range(KH):
            for kw in range(KW):
                o = o0 + kh * Wp + kw
                if windowed:
                    pieces.append(xrs[kd][0, 0, pl.ds(o, M), :])
                else:
                    pieces.append(xrs[0][0, d + dz0 + kd, pl.ds(o, M), :])
    patch = jnp.concatenate(pieces, axis=-1)
    y = jnp.dot(patch, wr[...], preferred_element_type=jnp.float32,
                precision=precision)
    y = y + br[...]
    if act == "relu":
        y = jnp.maximum(y, 0.0)
    elif act == "sigmoid":
        y = jax.nn.sigmoid(y)
    outr[0, 0] = y.astype(outr.dtype)


def _conv_stage(xp, wmat, bias, *, KD, KH, KW, Wp, H_out, D_out, act,
                dz0=0, o0=0, windowed=False, precision=None,
                compute_dtype=None, out_dtype=jnp.float32):
    if compute_dtype is not None:
        xp = xp.astype(compute_dtype)
        wmat = wmat.astype(compute_dtype)
    N, Dp, HWpad, Cin = xp.shape
    Ktot, Cout = wmat.shape
    M = H_out * Wp
    body = functools.partial(_conv_body, KD=KD, KH=KH, KW=KW, Wp=Wp, M=M,
                             dz0=dz0, o0=o0, act=act, windowed=windowed,
                             precision=precision)
    if windowed:
        x_specs = [
            pl.BlockSpec((1, 1, HWpad, Cin),
                         lambda n, d, kd=kd: (n, d + dz0 + kd, 0, 0))
            for kd in range(KD)
        ]
        x_args = (xp,) * KD
    else:
        x_specs = [pl.BlockSpec((1, Dp, HWpad, Cin), lambda n, d: (n, 0, 0, 0))]
        x_args = (xp,)
    return pl.pallas_call(
        body,
        grid=(N, D_out),
        in_specs=x_specs + [
            pl.BlockSpec((Ktot, Cout), lambda n, d: (0, 0)),
            pl.BlockSpec((1, Cout), lambda n, d: (0, 0)),
        ],
        out_specs=pl.BlockSpec((1, 1, M, Cout), lambda n, d: (n, d, 0, 0)),
        out_shape=jax.ShapeDtypeStruct((N, D_out, M, Cout), out_dtype),
    )(*x_args, wmat, bias)


def _pad_flat(x, pad):
    N, D, H, W, C = x.shape
    xp = jnp.pad(x, ((0, 0), (pad, pad), (pad, pad), (pad, pad), (0, 0)))
    Dp, Hp, Wpd = D + 2 * pad, H + 2 * pad, W + 2 * pad
    xp = xp.reshape(N, Dp, Hp * Wpd, C)
    return jnp.pad(xp, ((0, 0), (0, 0), (0, 8), (0, 0))), Wpd


def _unflat(y, H_out, W_out):
    N, D, M, C = y.shape
    Wp = M // H_out
    return y.reshape(N, D, H_out, Wp, C)[:, :, :, :W_out, :]


def _bn_fold(w_mat, conv_b, g, b, m, v, eps=1e-5):
    s = g / jnp.sqrt(v + eps)
    return w_mat * s[None, :], ((conv_b - m) * s + b)[None, :]


def _vq_body(zr, er, e2r, idxr, cntr, csumr, perpr, *, R, K, B, D):
    i = pl.program_id(0)
    z = zr[0]
    scores = jnp.dot(z, er[...], preferred_element_type=jnp.float32)
    z2 = jnp.sum(z * z, axis=1)
    t = (z2[:, None] + e2r[...]) - 2.0 * scores
    idx = jnp.argmin(t, axis=1).astype(jnp.int32)
    csum = jnp.sum(jnp.min(t, axis=1)).reshape(1, 1)
    idxr[0, 0] = idx
    onehot = (lax.broadcasted_iota(jnp.int32, (R, K), 1)
              == idx[:, None]).astype(jnp.float32)
    c = jnp.sum(onehot, axis=0)

    @pl.when(i == 0)
    def _():
        cntr[0] = c
        csumr[...] = csum

    @pl.when(i > 0)
    def _():
        cntr[0] += c
        csumr[...] = csumr[...] + csum

    @pl.when(i == pl.num_programs(0) - 1)
    def _():
        p = cntr[0] * (1.0 / B)
        perpr[...] = jnp.exp(-jnp.sum(p * jnp.log(p + 1e-10))).reshape(1, 1)
        csumr[...] = csumr[...] * (0.25 / (B * D))


def _vq_stage(flat, emb):
    B, D = flat.shape
    K = emb.shape[0]
    R = 2048
    e2 = jnp.sum(emb * emb, axis=1)[None, :]
    body = functools.partial(_vq_body, R=R, K=K, B=B, D=D)
    idx, cnt, csum, perp = pl.pallas_call(
        body,
        grid=(B // R,),
        in_specs=[
            pl.BlockSpec((1, R, D), lambda i: (i, 0, 0)),
            pl.BlockSpec((D, K), lambda i: (0, 0)),
            pl.BlockSpec((1, K), lambda i: (0, 0)),
        ],
        out_specs=[
            pl.BlockSpec((1, 1, R), lambda i: (i, 0, 0)),
            pl.BlockSpec((1, K), lambda i: (0, 0)),
            pl.BlockSpec((1, 1), lambda i: (0, 0)),
            pl.BlockSpec((1, 1), lambda i: (0, 0)),
        ],
        out_shape=[
            jax.ShapeDtypeStruct((B // R, 1, R), jnp.int32),
            jax.ShapeDtypeStruct((1, K), jnp.float32),
            jax.ShapeDtypeStruct((1, 1), jnp.float32),
            jax.ShapeDtypeStruct((1, 1), jnp.float32),
        ],
    )(flat.reshape(B // R, R, D), emb.T, e2)
    return idx.reshape(B), csum[0, 0], perp[0, 0]


def _quant_gather(emb, idx):
    emb = jnp.pad(emb, ((0, 0), (0, 128 - emb.shape[1])))
    B = idx.shape[0]
    D = emb.shape[1]
    info = plsc.get_sparse_core_info()
    NW = info.num_cores * info.num_subcores
    b_per_w = B // NW
    CH = 512
    n_ch = b_per_w // CH
    mesh = plsc.VectorSubcoreMesh(core_axis_name="c", subcore_axis_name="s")

    @functools.partial(
        pl.kernel, mesh=mesh,
        out_type=jax.ShapeDtypeStruct((B, D), jnp.float32),
        scratch_types=[
            pltpu.VMEM((CH,), jnp.int32),
            pltpu.VMEM((CH, D), jnp.float32),
            pltpu.SemaphoreType.DMA,
        ],
    )
    def gather_k(table_hbm, idx_hbm, out_hbm, idx_v, rows_v, sem):
        wid = lax.axis_index("s") * info.num_cores + lax.axis_index("c")
        base = wid * b_per_w
        for ch in range(n_ch):
            pltpu.sync_copy(idx_hbm.at[pl.ds(base + ch * CH, CH)], idx_v)
            pltpu.async_copy(table_hbm.at[idx_v], rows_v, sem).wait()
            pltpu.sync_copy(rows_v, out_hbm.at[pl.ds(base + ch * CH, CH)])

    return gather_k(emb, idx)


def kernel(x, p):
    N = x.shape[0]
    f32 = jnp.float32

    xp = jnp.pad(x[:, 0], ((0, 0), (1, 1), (1, 1), (1, 1)))
    xp = xp.reshape(N, 17, 2, 33, 2, 33, 2).transpose(0, 1, 3, 5, 2, 4, 6)
    xp = xp.reshape(N, 17, 33 * 33, 8)
    xp = jnp.pad(xp, ((0, 0), (0, 0), (0, 8), (0, 0)))
    w1 = p["ew1"].reshape(32, 2, 2, 2, 2, 2, 2).transpose(1, 3, 5, 2, 4, 6, 0)
    w1 = w1.reshape(64, 32)
    w1, b1 = _bn_fold(w1, p["eb1"], p["eg1"], p["ebt1"], p["em1"], p["ev1"])
    h = _conv_stage(xp, w1, b1, KD=2, KH=2, KW=2, Wp=33, H_out=32, D_out=16,
                    act="relu", compute_dtype=jnp.bfloat16,
                    out_dtype=jnp.bfloat16)
    h = _unflat(h, 32, 32)

    hp, Wp = _pad_flat(h, 1)
    w2 = p["ew2"].transpose(2, 3, 4, 1, 0).reshape(27 * 32, 64)
    w2, b2 = _bn_fold(w2, p["eb2"], p["eg2"], p["ebt2"], p["em2"], p["ev2"])
    h = _conv_stage(hp, w2, b2, KD=3, KH=3, KW=3, Wp=Wp, H_out=32, D_out=16,
                    act="relu", compute_dtype=jnp.bfloat16,
                    out_dtype=jnp.bfloat16)
    h = _unflat(h, 32, 32)

    hp, Wp = _pad_flat(h, 1)
    w3 = p["ew3"].transpose(2, 3, 4, 1, 0).reshape(27 * 64, 64)
    b3 = p["eb3"][None, :]
    z = _conv_stage(hp, w3, b3, KD=3, KH=3, KW=3, Wp=Wp, H_out=32, D_out=16,
                    act="none", compute_dtype=jnp.bfloat16)
    z = _unflat(z, 32, 32)
    flat = z.reshape(-1, 64)

    idx, commitment, perplexity = _vq_stage(flat, p["emb"])

    quant = _quant_gather(p["emb"], idx)[:, :64]
    quant = quant.reshape(N, 16, 32, 32, 64)

    hp, Wp = _pad_flat(quant.astype(jnp.bfloat16), 1)
    dw1 = p["dw1"].transpose(2, 3, 4, 1, 0).reshape(27 * 64, 64)
    dw1, db1 = _bn_fold(dw1, p["db1"], p["dg1"], p["dbt1"], p["dm1"], p["dv1"])
    h = _conv_stage(hp, dw1, db1, KD=3, KH=3, KW=3, Wp=Wp, H_out=32, D_out=16,
                    act="relu", compute_dtype=jnp.bfloat16,
                    out_dtype=jnp.bfloat16)
    h = _unflat(h, 32, 32)

    hp, Wp = _pad_flat(h, 1)
    f = jnp.flip(p["dwt"], axis=(2, 3, 4))
    f = f.reshape(64, 32, 2, 2, 2, 2, 2, 2).transpose(3, 5, 7, 2, 4, 6, 0, 1)
    f = f.reshape(8, 8 * 64, 32)
    s2 = p["dg2"] / jnp.sqrt(p["dv2"] + 1e-5)
    bt = ((p["dtb"] - p["dm2"]) * s2 + p["dbt2"])[None, :]
    phases = []
    for pz in range(2):
        for py in range(2):
            for px in range(2):
                ph = (pz * 2 + py) * 2 + px
                wp_ = f[ph] * s2[None, :]
                o = _conv_stage(hp, wp_, bt, KD=2, KH=2, KW=2, Wp=Wp,
                                H_out=32, D_out=16, act="relu",
                                dz0=pz, o0=py * Wp + px,
                                compute_dtype=jnp.bfloat16,
                                out_dtype=jnp.bfloat16)
                phases.append(_unflat(o, 32, 32))
    ph = jnp.stack(phases).reshape(2, 2, 2, N, 16, 32, 32, 32)
    h = ph.transpose(3, 4, 0, 5, 1, 6, 2, 7).reshape(N, 32, 64, 64, 32)

    hp, Wp = _pad_flat(h, 1)
    dw3 = p["dw3"].transpose(2, 3, 4, 1, 0).reshape(27 * 32, 1)
    dw3 = jnp.pad(dw3, ((0, 0), (0, 7)))
    db3 = jnp.pad(p["db3"], (0, 7))[None, :]
    y = _conv_stage(hp, dw3, db3, KD=3, KH=3, KW=3, Wp=Wp, H_out=64, D_out=32,
                    act="sigmoid", windowed=True,
                    compute_dtype=jnp.bfloat16)
    y = _unflat(y, 64, 64)[..., :1]
    x_recon = y.transpose(0, 4, 1, 2, 3).astype(f32)

    return x_recon, commitment.astype(f32), perplexity.astype(f32)


# speedup vs baseline: 2.6496x; 2.1663x over previous
import functools

import jax
import jax.numpy as jnp
import numpy as np
from jax import lax
from jax.experimental import pallas as pl
from jax.experimental.pallas import tpu as pltpu
from jax.experimental.pallas import tpu_sc as plsc

BF16 = jnp.bfloat16


def _conv_body(*args, KD, KH, KW, Wp, M, W_out, act, pad_out, phase_grid):
    xr, wr, br, outr = args
    d = pl.program_id(1)
    if phase_grid:
        pz = pl.program_id(2)
    base = d - 1 if pad_out else d

    def compute():
        pieces = []
        for kd in range(KD):
            dz = base + kd + (pz if phase_grid else 0)
            for kh in range(KH):
                for kw in range(KW):
                    pieces.append(xr[0, dz, pl.ds(kh * Wp + kw, M), :])
        patch = jnp.concatenate(pieces, axis=-1)
        y = jnp.dot(patch, wr[0] if phase_grid else wr[...],
                    preferred_element_type=jnp.float32)
        y = y + br[...]
        if act == "relu":
            y = jnp.maximum(y, 0.0)
        elif act == "sigmoid":
            y = jax.nn.sigmoid(y)
        return y

    if not pad_out:
        outr[0, 0] = compute().astype(outr.dtype)
        return

    D_in = pl.num_programs(1) - 2
    interior = (d >= 1) & (d <= D_in)
    R = outr.shape[2]
    Cout = outr.shape[3]

    @pl.when(interior)
    def _():
        y = compute()
        col = lax.broadcasted_iota(jnp.int32, (M, 1), 0) % Wp
        y = jnp.where(col < W_out, y, 0.0).astype(outr.dtype)
        outr[0, 0, : Wp + 1, :] = jnp.zeros((Wp + 1, Cout), outr.dtype)
        outr[0, 0, pl.ds(Wp + 1, M), :] = y
        outr[0, 0, pl.ds(Wp + 1 + M, R - Wp - 1 - M), :] = jnp.zeros(
            (R - Wp - 1 - M, Cout), outr.dtype)

    @pl.when(~interior)
    def _():
        outr[0, 0] = jnp.zeros((R, Cout), outr.dtype)


def _conv_stage(xp, wmat, bias, *, KD, KH, KW, Wp, H_out, W_out, D_out, act,
                pad_out, out_dtype=BF16, phase_grid=False):
    xp = xp.astype(BF16)
    wmat = wmat.astype(BF16)
    N, Dp, HWpad, Cin = xp.shape
    Cout = wmat.shape[-1]
    M = H_out * Wp
    body = functools.partial(_conv_body, KD=KD, KH=KH, KW=KW, Wp=Wp, M=M,
                             W_out=W_out, act=act, pad_out=pad_out,
                             phase_grid=phase_grid)
    if pad_out:
        assert Wp == W_out + 2
        R = (H_out + 2) * Wp + 8
        out_shape = jax.ShapeDtypeStruct(
            (N, D_out + 2, R, Cout * (2 if phase_grid else 1)), out_dtype)
        grid = (N, D_out + 2) + ((2,) if phase_grid else ())
    else:
        out_shape = jax.ShapeDtypeStruct((N, D_out, M, Cout), out_dtype)
        grid = (N, D_out)

    if phase_grid:
        in_specs = [
            pl.BlockSpec((1, Dp, HWpad, Cin), lambda n, d, z: (n, 0, 0, 0)),
            pl.BlockSpec((1,) + wmat.shape[1:], lambda n, d, z: (z, 0, 0)),
            pl.BlockSpec((1, Cout), lambda n, d, z: (0, 0)),
        ]
        out_specs = pl.BlockSpec((1, 1, R, Cout), lambda n, d, z: (n, d, 0, z))
    else:
        in_specs = [
            pl.BlockSpec((1, Dp, HWpad, Cin), lambda n, d: (n, 0, 0, 0)),
            pl.BlockSpec(wmat.shape, lambda n, d: (0, 0)),
            pl.BlockSpec((1, Cout), lambda n, d: (0, 0)),
        ]
        if pad_out:
            out_specs = pl.BlockSpec((1, 1, R, Cout), lambda n, d: (n, d, 0, 0))
        else:
            out_specs = pl.BlockSpec((1, 1, M, Cout), lambda n, d: (n, d, 0, 0))
    return pl.pallas_call(
        body, grid=grid, in_specs=in_specs, out_specs=out_specs,
        out_shape=out_shape,
    )(xp, wmat, bias)


def _bn_fold(w_mat, conv_b, g, b, m, v, eps=1e-5):
    s = g / jnp.sqrt(v + eps)
    return w_mat * s[None, :], ((conv_b - m) * s + b)[None, :]


def _vq_body(zr, er, e2r, idxr, cntr, csumr, perpr, *, R, K, TOT, D):
    i = pl.program_id(0)
    z = zr[0]
    scores = jnp.dot(z, er[...], preferred_element_type=jnp.float32)
    z2 = jnp.sum(z * z, axis=1)
    t = (z2[:, None] + e2r[...]) - 2.0 * scores
    idx = jnp.argmin(t, axis=1).astype(jnp.int32)
    csum = jnp.sum(jnp.min(t, axis=1)).reshape(1, 1)
    idxr[0, 0] = idx
    onehot = (lax.broadcasted_iota(jnp.int32, (R, K), 1)
              == idx[:, None]).astype(jnp.float32)
    c = jnp.sum(onehot, axis=0)

    @pl.when(i == 0)
    def _():
        cntr[0] = c
        csumr[...] = csum

    @pl.when(i > 0)
    def _():
        cntr[0] += c
        csumr[...] = csumr[...] + csum

    @pl.when(i == pl.num_programs(0) - 1)
    def _():
        p = cntr[0] * (1.0 / TOT)
        perpr[...] = jnp.exp(-jnp.sum(p * jnp.log(p + 1e-10))).reshape(1, 1)
        csumr[...] = csumr[...] * (0.25 / (TOT * D))


def _vq_stage(flat, emb):
    B, D = flat.shape
    K = emb.shape[0]
    R = 2048
    e2 = jnp.sum(emb * emb, axis=1)[None, :]
    body = functools.partial(_vq_body, R=R, K=K, TOT=B, D=D)
    idx, cnt, csum, perp = pl.pallas_call(
        body,
        grid=(B // R,),
        in_specs=[
            pl.BlockSpec((1, R, D), lambda i: (i, 0, 0)),
            pl.BlockSpec((D, K), lambda i: (0, 0)),
            pl.BlockSpec((1, K), lambda i: (0, 0)),
        ],
        out_specs=[
            pl.BlockSpec((1, 1, R), lambda i: (i, 0, 0)),
            pl.BlockSpec((1, K), lambda i: (0, 0)),
            pl.BlockSpec((1, 1), lambda i: (0, 0)),
            pl.BlockSpec((1, 1), lambda i: (0, 0)),
        ],
        out_shape=[
            jax.ShapeDtypeStruct((B // R, 1, R), jnp.int32),
            jax.ShapeDtypeStruct((1, K), jnp.float32),
            jax.ShapeDtypeStruct((1, 1), jnp.float32),
            jax.ShapeDtypeStruct((1, 1), jnp.float32),
        ],
    )(flat.reshape(B // R, R, D), emb.T, e2)
    return idx.reshape(B), csum[0, 0], perp[0, 0]


def _quant_gather(emb, idx):
    emb = jnp.pad(emb, ((0, 0), (0, 128 - emb.shape[1])))
    B = idx.shape[0]
    D = emb.shape[1]
    info = plsc.get_sparse_core_info()
    NW = info.num_cores * info.num_subcores
    b_per_w = B // NW
    CH = 512
    n_ch = b_per_w // CH
    mesh = plsc.VectorSubcoreMesh(core_axis_name="c", subcore_axis_name="s")

    @functools.partial(
        pl.kernel, mesh=mesh,
        out_type=jax.ShapeDtypeStruct((B, D), jnp.float32),
        scratch_types=[
            pltpu.VMEM((CH,), jnp.int32),
            pltpu.VMEM((CH, D), jnp.float32),
            pltpu.SemaphoreType.DMA,
        ],
    )
    def gather_k(table_hbm, idx_hbm, out_hbm, idx_v, rows_v, sem):
        wid = lax.axis_index("s") * info.num_cores + lax.axis_index("c")
        base = wid * b_per_w
        for ch in range(n_ch):
            pltpu.sync_copy(idx_hbm.at[pl.ds(base + ch * CH, CH)], idx_v)
            pltpu.async_copy(table_hbm.at[idx_v], rows_v, sem).wait()
            pltpu.sync_copy(rows_v, out_hbm.at[pl.ds(base + ch * CH, CH)])

    return gather_k(emb, idx)


def _w_e1(p):
    w1 = p["ew1"].reshape(32, 2, 2, 2, 2, 2, 2).transpose(1, 3, 5, 2, 4, 6, 0)
    w1 = w1.reshape(64, 32)
    return _bn_fold(w1, p["eb1"], p["eg1"], p["ebt1"], p["em1"], p["ev1"])


def _w_convT(p):
    f = jnp.flip(p["dwt"], axis=(2, 3, 4))
    F = f.reshape(64, 32, 2, 2, 2, 2, 2, 2).transpose(3, 5, 7, 2, 4, 6, 0, 1)
    s2 = p["dg2"] / jnp.sqrt(p["dv2"] + 1e-5)
    F = F * s2[None, None, None, None, None, None, None, :]
    T = np.zeros((3, 2, 2), np.float32)
    for ph in range(2):
        for k in range(2):
            T[ph + k, ph, k] = 1.0
    T = jnp.asarray(T)
    W = jnp.einsum("hyk,wxl,zyxAklEC->zAhwEyxC", T, T, F)
    W = W.reshape(2, 2 * 3 * 3 * 64, 2 * 2 * 32)
    bt = ((p["dtb"] - p["dm2"]) * s2 + p["dbt2"])
    bias = jnp.tile(bt, (4,))[None, :]
    return W, bias


def _w_d3(p):
    S = np.zeros((2, 3, 2, 3), np.float32)
    for q in range(2):
        for kd in range(3):
            for pz in range(2):
                j = 2 * (kd - 1) + pz + 1 - q
                if 0 <= j <= 2:
                    S[q, kd, pz, j] = 1.0
    S = jnp.asarray(S)
    w = p["dw3"][0]
    W = jnp.einsum("aKZj,bLYk,cMXl,Djkl->KLMZYXDabc", S, S, S, w)
    W = W.reshape(27 * 256, 8)
    bias = jnp.full((1, 8), p["db3"][0])
    return W, bias


def kernel(x, p):
    N = x.shape[0]

    xp = jnp.pad(x[:, 0], ((0, 0), (1, 1), (1, 1), (1, 3)))
    xp = xp.reshape(N, 17, 2, 33, 2, 34, 2).transpose(0, 1, 3, 5, 2, 4, 6)
    xp = xp.reshape(N, 17, 33 * 34, 8)
    xp = jnp.pad(xp, ((0, 0), (0, 0), (0, 8), (0, 0)))
    w1, b1 = _w_e1(p)
    h = _conv_stage(xp, w1, b1, KD=2, KH=2, KW=2, Wp=34, H_out=32, W_out=32,
                    D_out=16, act="relu", pad_out=True)

    w2 = p["ew2"].transpose(2, 3, 4, 1, 0).reshape(27 * 32, 64)
    w2, b2 = _bn_fold(w2, p["eb2"], p["eg2"], p["ebt2"], p["em2"], p["ev2"])
    h = _conv_stage(h, w2, b2, KD=3, KH=3, KW=3, Wp=34, H_out=32, W_out=32,
                    D_out=16, act="relu", pad_out=True)

    w3 = p["ew3"].transpose(2, 3, 4, 1, 0).reshape(27 * 64, 64)
    b3 = p["eb3"][None, :]
    z = _conv_stage(h, w3, b3, KD=3, KH=3, KW=3, Wp=34, H_out=32, W_out=32,
                    D_out=16, act="none", pad_out=False,
                    out_dtype=jnp.float32)
    flat = z.reshape(N, 16, 32, 34, 64)[:, :, :, :32, :].reshape(-1, 64)

    idx, commitment, perplexity = _vq_stage(flat, p["emb"])

    quant = _quant_gather(p["emb"], idx)[:, :64]
    quant = quant.reshape(N, 16, 32, 32, 64)

    qp = jnp.pad(quant.astype(BF16),
                 ((0, 0), (1, 1), (1, 1), (1, 1), (0, 0)))
    qp = qp.reshape(N, 18, 34 * 34, 64)
    qp = jnp.pad(qp, ((0, 0), (0, 0), (0, 8), (0, 0)))
    dw1 = p["dw1"].transpose(2, 3, 4, 1, 0).reshape(27 * 64, 64)
    dw1, db1 = _bn_fold(dw1, p["db1"], p["dg1"], p["dbt1"], p["dm1"], p["dv1"])
    h = _conv_stage(qp, dw1, db1, KD=3, KH=3, KW=3, Wp=34, H_out=32, W_out=32,
                    D_out=16, act="relu", pad_out=True)

    Wt, bt = _w_convT(p)
    h = _conv_stage(h, Wt, bt, KD=2, KH=3, KW=3, Wp=34, H_out=32, W_out=32,
                    D_out=16, act="relu", pad_out=True,
                    phase_grid=True)

    W3, b3d = _w_d3(p)
    y = _conv_stage(h, W3, b3d, KD=3, KH=3, KW=3, Wp=34, H_out=32, W_out=32,
                    D_out=16, act="sigmoid", pad_out=False,
                    out_dtype=jnp.float32)

    y = y.reshape(N, 16, 32, 34, 8)[:, :, :, :32, :]
    y = y.reshape(N, 16, 32, 32, 2, 2, 2).transpose(0, 1, 4, 2, 5, 3, 6)
    x_recon = y.reshape(N, 1, 32, 64, 64)

    return x_recon, commitment.astype(jnp.float32), perplexity.astype(jnp.float32)


# speedup vs baseline: 2.8164x; 1.0629x over previous
import functools

import jax
import jax.numpy as jnp
import numpy as np
from jax import lax
from jax.experimental import pallas as pl
from jax.experimental.pallas import tpu as pltpu
from jax.experimental.pallas import tpu_sc as plsc

BF16 = jnp.bfloat16


def _conv_body(*args, KD, KH, KW, Wp, M, W_out, act, pad_out, phase_grid):
    xr, wr, br, outr = args
    if phase_grid:
        pz = pl.program_id(1)
        d = pl.program_id(2)
    else:
        d = pl.program_id(1)
    base = d - 1 if pad_out else d

    def compute():
        pieces = []
        for kd in range(KD):
            dz = base + kd + (pz if phase_grid else 0)
            for kh in range(KH):
                for kw in range(KW):
                    pieces.append(xr[0, dz, pl.ds(kh * Wp + kw, M), :])
        patch = jnp.concatenate(pieces, axis=-1)
        y = jnp.dot(patch, wr[0] if phase_grid else wr[...],
                    preferred_element_type=jnp.float32)
        y = y + br[...]
        if act == "relu":
            y = jnp.maximum(y, 0.0)
        elif act == "sigmoid":
            y = jax.nn.sigmoid(y)
        return y

    if not pad_out:
        outr[0, 0] = compute().astype(outr.dtype)
        return

    D_in = pl.num_programs(2 if phase_grid else 1) - 2
    interior = (d >= 1) & (d <= D_in)
    R = outr.shape[2]
    Cout = outr.shape[3]

    @pl.when(interior)
    def _():
        y = compute()
        col = lax.broadcasted_iota(jnp.int32, (M, 1), 0) % Wp
        y = jnp.where(col < W_out, y, 0.0).astype(outr.dtype)
        outr[0, 0, : Wp + 1, :] = jnp.zeros((Wp + 1, Cout), outr.dtype)
        outr[0, 0, pl.ds(Wp + 1, M), :] = y
        outr[0, 0, pl.ds(Wp + 1 + M, R - Wp - 1 - M), :] = jnp.zeros(
            (R - Wp - 1 - M, Cout), outr.dtype)

    @pl.when(~interior)
    def _():
        outr[0, 0] = jnp.zeros((R, Cout), outr.dtype)


def _conv_stage(xp, wmat, bias, *, KD, KH, KW, Wp, H_out, W_out, D_out, act,
                pad_out, out_dtype=BF16, phase_grid=False):
    xp = xp.astype(BF16)
    wmat = wmat.astype(BF16)
    N, Dp, HWpad, Cin = xp.shape
    Cout = wmat.shape[-1]
    M = H_out * Wp
    body = functools.partial(_conv_body, KD=KD, KH=KH, KW=KW, Wp=Wp, M=M,
                             W_out=W_out, act=act, pad_out=pad_out,
                             phase_grid=phase_grid)
    if pad_out:
        assert Wp == W_out + 2
        R = (H_out + 2) * Wp + 8
        out_shape = jax.ShapeDtypeStruct(
            (N, D_out + 2, R, Cout * (2 if phase_grid else 1)), out_dtype)
        grid = (N, 2, D_out + 2) if phase_grid else (N, D_out + 2)
    else:
        out_shape = jax.ShapeDtypeStruct((N, D_out, M, Cout), out_dtype)
        grid = (N, D_out)

    if phase_grid:
        in_specs = [
            pl.BlockSpec((1, Dp, HWpad, Cin), lambda n, z, d: (n, 0, 0, 0)),
            pl.BlockSpec((1,) + wmat.shape[1:], lambda n, z, d: (z, 0, 0)),
            pl.BlockSpec((1, Cout), lambda n, z, d: (0, 0)),
        ]
        out_specs = pl.BlockSpec((1, 1, R, Cout), lambda n, z, d: (n, d, 0, z))
    else:
        in_specs = [
            pl.BlockSpec((1, Dp, HWpad, Cin), lambda n, d: (n, 0, 0, 0)),
            pl.BlockSpec(wmat.shape, lambda n, d: (0, 0)),
            pl.BlockSpec((1, Cout), lambda n, d: (0, 0)),
        ]
        if pad_out:
            out_specs = pl.BlockSpec((1, 1, R, Cout), lambda n, d: (n, d, 0, 0))
        else:
            out_specs = pl.BlockSpec((1, 1, M, Cout), lambda n, d: (n, d, 0, 0))
    return pl.pallas_call(
        body, grid=grid, in_specs=in_specs, out_specs=out_specs,
        out_shape=out_shape,
    )(xp, wmat, bias)


def _bn_fold(w_mat, conv_b, g, b, m, v, eps=1e-5):
    s = g / jnp.sqrt(v + eps)
    return w_mat * s[None, :], ((conv_b - m) * s + b)[None, :]


def _vq_body(zr, er, e2r, idxr, cntr, csumr, perpr, *, R, K, TOT, D, Wp,
             W_out):
    i = pl.program_id(0)
    z = zr[0]
    scores = jnp.dot(z, er[...], preferred_element_type=jnp.float32)
    z2 = jnp.sum(z * z, axis=1)
    t = (z2[:, None] + e2r[...]) - 2.0 * scores
    idx = jnp.argmin(t, axis=1).astype(jnp.int32)
    valid = (lax.broadcasted_iota(jnp.int32, (R,), 0) % Wp) < W_out
    idx = jnp.where(valid, idx, K)
    csum = jnp.sum(jnp.where(valid, jnp.min(t, axis=1), 0.0)).reshape(1, 1)
    idxr[0, 0] = idx
    onehot = (lax.broadcasted_iota(jnp.int32, (R, K), 1)
              == idx[:, None]).astype(jnp.float32)
    c = jnp.sum(onehot, axis=0)

    @pl.when(i == 0)
    def _():
        cntr[0] = c
        csumr[...] = csum

    @pl.when(i > 0)
    def _():
        cntr[0] += c
        csumr[...] = csumr[...] + csum

    @pl.when(i == pl.num_programs(0) - 1)
    def _():
        p = cntr[0] * (1.0 / TOT)
        perpr[...] = jnp.exp(-jnp.sum(p * jnp.log(p + 1e-10))).reshape(1, 1)
        csumr[...] = csumr[...] * (0.25 / (TOT * D))


def _vq_stage(z, emb, TOT, Wp, W_out):
    N_, DP, R, D = z.shape
    flat = z.reshape(N_ * DP, R, D)
    B = N_ * DP * R
    K = emb.shape[0]
    e2 = jnp.sum(emb * emb, axis=1)[None, :]
    body = functools.partial(_vq_body, R=R, K=K, TOT=TOT, D=D, Wp=Wp,
                             W_out=W_out)
    idx, cnt, csum, perp = pl.pallas_call(
        body,
        grid=(B // R,),
        in_specs=[
            pl.BlockSpec((1, R, D), lambda i: (i, 0, 0)),
            pl.BlockSpec((D, K), lambda i: (0, 0)),
            pl.BlockSpec((1, K), lambda i: (0, 0)),
        ],
        out_specs=[
            pl.BlockSpec((1, 1, R), lambda i: (i, 0, 0)),
            pl.BlockSpec((1, K), lambda i: (0, 0)),
            pl.BlockSpec((1, 1), lambda i: (0, 0)),
            pl.BlockSpec((1, 1), lambda i: (0, 0)),
        ],
        out_shape=[
            jax.ShapeDtypeStruct((B // R, 1, R), jnp.int32),
            jax.ShapeDtypeStruct((1, K), jnp.float32),
            jax.ShapeDtypeStruct((1, 1), jnp.float32),
            jax.ShapeDtypeStruct((1, 1), jnp.float32),
        ],
    )(flat, emb.T, e2)
    return idx.reshape(B), csum[0, 0], perp[0, 0]


def _quant_gather(emb, idx):
    emb = jnp.pad(emb, ((0, 8), (0, 128 - emb.shape[1])))
    B = idx.shape[0]
    D = emb.shape[1]
    info = plsc.get_sparse_core_info()
    NW = info.num_cores * info.num_subcores
    b_per_w = B // NW
    n_ch = 8
    CH = b_per_w // n_ch
    mesh = plsc.VectorSubcoreMesh(core_axis_name="c", subcore_axis_name="s")

    @functools.partial(
        pl.kernel, mesh=mesh,
        out_type=jax.ShapeDtypeStruct((B, D), jnp.float32),
        scratch_types=[
            [pltpu.VMEM((CH,), jnp.int32) for _ in range(n_ch)],
            [pltpu.VMEM((CH, D), jnp.float32) for _ in range(2)],
            [pltpu.SemaphoreType.DMA for _ in range(2)],
        ],
    )
    def gather_k(table_hbm, idx_hbm, out_hbm, idx_v, rows_v, sem):
        wid = lax.axis_index("s") * info.num_cores + lax.axis_index("c")
        base = wid * b_per_w
        for ch in range(n_ch):
            pltpu.sync_copy(idx_hbm.at[pl.ds(base + ch * CH, CH)], idx_v[ch])
        pend = {0: pltpu.async_copy(table_hbm.at[idx_v[0]], rows_v[0], sem[0])}
        for ch in range(n_ch):
            if ch + 1 < n_ch:
                pend[ch + 1] = pltpu.async_copy(
                    table_hbm.at[idx_v[ch + 1]], rows_v[(ch + 1) % 2],
                    sem[(ch + 1) % 2])
            pend[ch].wait()
            pltpu.sync_copy(rows_v[ch % 2],
                            out_hbm.at[pl.ds(base + ch * CH, CH)])

    return gather_k(emb, idx)


def _w_e1(p):
    w1 = p["ew1"].reshape(32, 2, 2, 2, 2, 2, 2).transpose(1, 3, 5, 2, 4, 6, 0)
    w1 = w1.reshape(64, 32)
    return _bn_fold(w1, p["eb1"], p["eg1"], p["ebt1"], p["em1"], p["ev1"])


def _w_convT(p):
    f = jnp.flip(p["dwt"], axis=(2, 3, 4))
    F = f.reshape(64, 32, 2, 2, 2, 2, 2, 2).transpose(3, 5, 7, 2, 4, 6, 0, 1)
    s2 = p["dg2"] / jnp.sqrt(p["dv2"] + 1e-5)
    F = F * s2[None, None, None, None, None, None, None, :]
    T = np.zeros((3, 2, 2), np.float32)
    for ph in range(2):
        for k in range(2):
            T[ph + k, ph, k] = 1.0
    T = jnp.asarray(T)
    W = jnp.einsum("hyk,wxl,zyxAklEC->zAhwEyxC", T, T, F)
    W = W.reshape(2, 2 * 3 * 3 * 64, 2 * 2 * 32)
    bt = ((p["dtb"] - p["dm2"]) * s2 + p["dbt2"])
    bias = jnp.tile(bt, (4,))[None, :]
    return W, bias


def _w_d3(p):
    S = np.zeros((2, 3, 2, 3), np.float32)
    for q in range(2):
        for kd in range(3):
            for pz in range(2):
                j = 2 * (kd - 1) + pz + 1 - q
                if 0 <= j <= 2:
                    S[q, kd, pz, j] = 1.0
    S = jnp.asarray(S)
    w = p["dw3"][0]
    W = jnp.einsum("aKZj,bLYk,cMXl,Djkl->KLMZYXDabc", S, S, S, w)
    W = W.reshape(27 * 256, 8)
    bias = jnp.full((1, 8), p["db3"][0])
    return W, bias


def kernel(x, p):
    N = x.shape[0]

    xp = jnp.pad(x[:, 0], ((0, 0), (1, 1), (1, 1), (1, 3)))
    xp = xp.reshape(N, 17, 2, 33, 2, 34, 2).transpose(0, 1, 3, 5, 2, 4, 6)
    xp = xp.reshape(N, 17, 33 * 34, 8)
    xp = jnp.pad(xp, ((0, 0), (0, 0), (0, 8), (0, 0)))
    w1, b1 = _w_e1(p)
    h = _conv_stage(xp, w1, b1, KD=2, KH=2, KW=2, Wp=34, H_out=32, W_out=32,
                    D_out=16, act="relu", pad_out=True)

    w2 = p["ew2"].transpose(2, 3, 4, 1, 0).reshape(27 * 32, 64)
    w2, b2 = _bn_fold(w2, p["eb2"], p["eg2"], p["ebt2"], p["em2"], p["ev2"])
    h = _conv_stage(h, w2, b2, KD=3, KH=3, KW=3, Wp=34, H_out=32, W_out=32,
                    D_out=16, act="relu", pad_out=True)

    w3 = p["ew3"].transpose(2, 3, 4, 1, 0).reshape(27 * 64, 64)
    b3 = p["eb3"][None, :]
    z = _conv_stage(h, w3, b3, KD=3, KH=3, KW=3, Wp=34, H_out=32, W_out=32,
                    D_out=16, act="none", pad_out=False,
                    out_dtype=jnp.float32)

    idx, commitment, perplexity = _vq_stage(z, p["emb"], TOT=N * 16 * 32 * 32,
                                            Wp=34, W_out=32)

    quant = _quant_gather(p["emb"], idx)[:, :64].astype(BF16)
    quant = quant.reshape(N, 16, 1088, 64)

    qp = jnp.pad(quant, ((0, 0), (1, 1), (35, 41), (0, 0)))
    dw1 = p["dw1"].transpose(2, 3, 4, 1, 0).reshape(27 * 64, 64)
    dw1, db1 = _bn_fold(dw1, p["db1"], p["dg1"], p["dbt1"], p["dm1"], p["dv1"])
    h = _conv_stage(qp, dw1, db1, KD=3, KH=3, KW=3, Wp=34, H_out=32, W_out=32,
                    D_out=16, act="relu", pad_out=True)

    Wt, bt = _w_convT(p)
    h = _conv_stage(h, Wt, bt, KD=2, KH=3, KW=3, Wp=34, H_out=32, W_out=32,
                    D_out=16, act="relu", pad_out=True,
                    phase_grid=True)

    W3, b3d = _w_d3(p)
    y = _conv_stage(h, W3, b3d, KD=3, KH=3, KW=3, Wp=34, H_out=32, W_out=32,
                    D_out=16, act="sigmoid", pad_out=False,
                    out_dtype=jnp.float32)

    y = y.reshape(N, 16, 32, 34, 8)[:, :, :, :32, :]
    y = y.reshape(N, 16, 32, 32, 2, 2, 2).transpose(0, 1, 4, 2, 5, 3, 6)
    x_recon = y.reshape(N, 1, 32, 64, 64)

    return x_recon, commitment.astype(jnp.float32), perplexity.astype(jnp.float32)
